# Initial kernel scaffold; baseline (speedup 1.0000x reference)
import jax
import jax.numpy as jnp
from jax.experimental import pallas as pl


def kernel(f_atoms, f_bonds, edge_index, graph_ids, W_i, b_i, W_h, b_h, W_o, b_o, W_f1, b_f1, W_f2, b_f2):
    raise NotImplementedError("write your pallas kernel here")


# speedup vs baseline: 3.3661x; 3.3661x over previous
import functools

import jax
import jax.numpy as jnp
from jax import lax
from jax.experimental import pallas as pl
from jax.experimental.pallas import tpu as pltpu
from jax.experimental.pallas import tpu_sc as plsc

N = 10000
E = 320000
DA = 128
DE = 16
H = 128
G = 64
NC = 2
NS = 16
NW = NC * NS
CHUNK = 128
ROWS = E // CHUNK
MAX_K = (ROWS + NW - 1) // NW
NP = 10240
NODE_PER_TILE = NP // NS


def _mm_body(x_ref, w_ref, b_ref, o_ref):
  o_ref[...] = (
      jnp.dot(x_ref[...], w_ref[...], preferred_element_type=jnp.float32)
      + b_ref[...]
  )


def _dense(x, w, b, block):
  m, k = x.shape
  n = w.shape[1]
  return pl.pallas_call(
      _mm_body,
      grid=(m // block,),
      in_specs=[
          pl.BlockSpec((block, k), lambda i: (i, 0)),
          pl.BlockSpec((k, n), lambda i: (0, 0)),
          pl.BlockSpec((1, n), lambda i: (0, 0)),
      ],
      out_specs=pl.BlockSpec((block, n), lambda i: (i, 0)),
      out_shape=jax.ShapeDtypeStruct((m, n), jnp.float32),
  )(x, w, b.reshape(1, n))


def _mm2_body(a_ref, w_ref, b_ref, o_ref):
  x = a_ref[0] + a_ref[1]
  o_ref[...] = (
      jnp.dot(x, w_ref[...], preferred_element_type=jnp.float32) + b_ref[...]
  )


def _dense_fold(acc, w, b, block):
  _, m, k = acc.shape
  n = w.shape[1]
  return pl.pallas_call(
      _mm2_body,
      grid=(m // block,),
      in_specs=[
          pl.BlockSpec((2, block, k), lambda i: (0, i, 0)),
          pl.BlockSpec((k, n), lambda i: (0, 0)),
          pl.BlockSpec((1, n), lambda i: (0, 0)),
      ],
      out_specs=pl.BlockSpec((block, n), lambda i: (i, 0)),
      out_shape=jax.ShapeDtypeStruct((m, k), jnp.float32),
  )(acc, w, b.reshape(1, n))


_FBLK = 2000
_FGRID = N // _FBLK


def _final_body(fa_ref, acc_ref, gid_ref, wot_ref, wob_ref, bo_ref,
                wf1_ref, bf1_ref, wf2_ref, bf2_ref, o_ref, mol_ref, cnt_ref):
  i = pl.program_id(0)

  @pl.when(i == 0)
  def _():
    mol_ref[...] = jnp.zeros_like(mol_ref)
    cnt_ref[...] = jnp.zeros_like(cnt_ref)

  a2 = acc_ref[0] + acc_ref[1]
  atom = jnp.maximum(
      jnp.dot(fa_ref[...], wot_ref[...], preferred_element_type=jnp.float32)
      + jnp.dot(a2, wob_ref[...], preferred_element_type=jnp.float32)
      + bo_ref[...],
      0.0,
  )
  gid = gid_ref[0, 0]
  onehot = (
      lax.broadcasted_iota(jnp.int32, (G, _FBLK), 0) == gid[None, :]
  ).astype(jnp.float32)
  mol_ref[...] += jnp.dot(onehot, atom, preferred_element_type=jnp.float32)
  cnt_ref[...] += jnp.sum(onehot, axis=1, keepdims=True)

  @pl.when(i == _FGRID - 1)
  def _():
    mol = mol_ref[...] / jnp.maximum(cnt_ref[...], 1.0)
    hid = jnp.maximum(
        jnp.dot(mol, wf1_ref[...], preferred_element_type=jnp.float32)
        + bf1_ref[...],
        0.0,
    )
    o_ref[...] = (
        jnp.dot(hid, wf2_ref[...], preferred_element_type=jnp.float32)
        + bf2_ref[...]
    )


def _final(f_atoms, acc, gid_row, w_o_t, w_o_b, b_o, w_f1, b_f1, w_f2, b_f2):
  t = w_f2.shape[1]
  return pl.pallas_call(
      _final_body,
      grid=(_FGRID,),
      in_specs=[
          pl.BlockSpec((_FBLK, DA), lambda i: (i, 0)),
          pl.BlockSpec((2, _FBLK, H), lambda i: (0, i, 0)),
          pl.BlockSpec((1, 1, _FBLK), lambda i: (i, 0, 0)),
          pl.BlockSpec((DA, H), lambda i: (0, 0)),
          pl.BlockSpec((H, H), lambda i: (0, 0)),
          pl.BlockSpec((1, H), lambda i: (0, 0)),
          pl.BlockSpec((H, H), lambda i: (0, 0)),
          pl.BlockSpec((1, H), lambda i: (0, 0)),
          pl.BlockSpec((H, t), lambda i: (0, 0)),
          pl.BlockSpec((1, t), lambda i: (0, 0)),
      ],
      out_specs=pl.BlockSpec((G, t), lambda i: (0, 0)),
      out_shape=jax.ShapeDtypeStruct((G, t), jnp.float32),
      scratch_shapes=[
          pltpu.VMEM((G, H), jnp.float32),
          pltpu.VMEM((G, 1), jnp.float32),
      ],
  )(f_atoms, acc, gid_row, w_o_t, w_o_b, b_o.reshape(1, H),
    w_f1, b_f1.reshape(1, H), w_f2, b_f2.reshape(1, t))


def _sweep_impl(write_h0, table, rows_in, srcr, dstr, zrows,
                accout, h0out, acc, src_v, dst_v, rows_v, gath_v, sem1, sem2):
  c = lax.axis_index("c")
  s = lax.axis_index("s")
  w = s * NC + c

  pltpu.sync_copy(zrows, acc.at[pl.ds(s * NODE_PER_TILE, NODE_PER_TILE)])
  plsc.subcore_barrier()

  def body(k, carry):
    r = w + NW * k

    @pl.when(r < ROWS)
    def _():
      pltpu.sync_copy(srcr.at[r], src_v)
      pltpu.sync_copy(dstr.at[pl.ds(r, 1)], dst_v)
      cp1 = pltpu.async_copy(rows_in.at[pl.ds(r * CHUNK, CHUNK)], rows_v, sem1)
      cp2 = pltpu.async_copy(table.at[src_v], gath_v, sem2)
      cp1.wait()
      cp2.wait()

      def crow(rr, cy):
        for j in range(H // 16):
          sl = pl.ds(j * 16, 16)
          gath_v[rr, sl] = jnp.maximum(gath_v[rr, sl] + rows_v[rr, sl], 0.0)
        return cy

      lax.fori_loop(0, CHUNK, crow, 0)
      if write_h0:
        pltpu.sync_copy(gath_v, h0out.at[pl.ds(r * CHUNK, CHUNK)])
      pltpu.sync_copy(gath_v, acc.at[dst_v.at[0]], add=True)

    return carry

  lax.fori_loop(0, MAX_K, body, 0)

  plsc.subcore_barrier()
  pltpu.sync_copy(
      acc.at[pl.ds(s * NODE_PER_TILE, NODE_PER_TILE)],
      accout.at[c, pl.ds(s * NODE_PER_TILE, NODE_PER_TILE)],
  )


def _sweep0_body(table, rows_in, srcr, dstr, zrows, accout, h0out,
                 acc, src_v, dst_v, rows_v, gath_v, sem1, sem2):
  _sweep_impl(True, table, rows_in, srcr, dstr, zrows, accout, h0out,
              acc, src_v, dst_v, rows_v, gath_v, sem1, sem2)


def _sweep_body(table, rows_in, srcr, dstr, zrows, accout,
                acc, src_v, dst_v, rows_v, gath_v, sem1, sem2):
  _sweep_impl(False, table, rows_in, srcr, dstr, zrows, accout, None,
              acc, src_v, dst_v, rows_v, gath_v, sem1, sem2)


_SC_SCRATCH = [
    pltpu.VMEM_SHARED((NP, H), jnp.float32),
    pltpu.VMEM((CHUNK,), jnp.int32),
    pltpu.VMEM((1, CHUNK), jnp.int32),
    pltpu.VMEM((CHUNK, H), jnp.float32),
    pltpu.VMEM((CHUNK, H), jnp.float32),
    pltpu.SemaphoreType.DMA,
    pltpu.SemaphoreType.DMA,
]

_MESH = plsc.VectorSubcoreMesh(core_axis_name="c", subcore_axis_name="s")

_sweep0 = pl.kernel(
    _sweep0_body,
    out_type=(
        jax.ShapeDtypeStruct((2, NP, H), jnp.float32),
        jax.ShapeDtypeStruct((E, H), jnp.float32),
    ),
    mesh=_MESH,
    scratch_types=_SC_SCRATCH,
)

_sweep = pl.kernel(
    _sweep_body,
    out_type=jax.ShapeDtypeStruct((2, NP, H), jnp.float32),
    mesh=_MESH,
    scratch_types=_SC_SCRATCH,
)


def kernel(f_atoms, f_bonds, edge_index, graph_ids,
           W_i, b_i, W_h, b_h, W_o, b_o, W_f1, b_f1, W_f2, b_f2):
  srcr = edge_index[0].astype(jnp.int32).reshape(ROWS, CHUNK)
  dstr = edge_index[1].astype(jnp.int32).reshape(ROWS, CHUNK)
  gid_row = graph_ids.astype(jnp.int32).reshape(_FGRID, 1, _FBLK)
  zrows = jnp.zeros((NODE_PER_TILE, H), jnp.float32)

  A = _dense(f_atoms, W_i[:DA], jnp.zeros((H,), jnp.float32), block=2000)
  B = _dense(f_bonds, W_i[DA:], b_i, block=2000)

  acc0, h0 = _sweep0(A, B, srcr, dstr, zrows)
  Q0 = _dense_fold(acc0, W_h, b_h, block=2000)
  acc1 = _sweep(Q0, h0, srcr, dstr, zrows)
  Q1 = _dense_fold(acc1, W_h, b_h, block=2000)
  acc2 = _sweep(Q1, h0, srcr, dstr, zrows)

  return _final(f_atoms, acc2, gid_row, W_o[:DA], W_o[DA:], b_o,
                W_f1, b_f1, W_f2, b_f2)


# speedup vs baseline: 4.7874x; 1.4223x over previous
import functools

import jax
import jax.numpy as jnp
from jax import lax
from jax.experimental import pallas as pl
from jax.experimental.pallas import tpu as pltpu
from jax.experimental.pallas import tpu_sc as plsc

N = 10000
E = 320000
DA = 128
DE = 16
H = 128
G = 64
NC = 2
NS = 16
NW = NC * NS
CHUNK = 80
ROWS = E // CHUNK
MAX_K = (ROWS + NW - 1) // NW
NP = 10240
NODE_PER_TILE = NP // NS


def _mm_body(x_ref, w_ref, b_ref, o_ref):
  o_ref[...] = (
      jnp.dot(x_ref[...], w_ref[...], preferred_element_type=jnp.float32)
      + b_ref[...]
  )


def _dense(x, w, b, block):
  m, k = x.shape
  n = w.shape[1]
  return pl.pallas_call(
      _mm_body,
      grid=(m // block,),
      in_specs=[
          pl.BlockSpec((block, k), lambda i: (i, 0)),
          pl.BlockSpec((k, n), lambda i: (0, 0)),
          pl.BlockSpec((1, n), lambda i: (0, 0)),
      ],
      out_specs=pl.BlockSpec((block, n), lambda i: (i, 0)),
      out_shape=jax.ShapeDtypeStruct((m, n), jnp.float32),
  )(x, w, b.reshape(1, n))


def _mm2_body(a_ref, w_ref, b_ref, o_ref):
  x = a_ref[0] + a_ref[1]
  o_ref[...] = (
      jnp.dot(x, w_ref[...], preferred_element_type=jnp.float32) + b_ref[...]
  )


def _dense_fold(acc, w, b, block):
  _, m, k = acc.shape
  n = w.shape[1]
  return pl.pallas_call(
      _mm2_body,
      grid=(m // block,),
      in_specs=[
          pl.BlockSpec((2, block, k), lambda i: (0, i, 0)),
          pl.BlockSpec((k, n), lambda i: (0, 0)),
          pl.BlockSpec((1, n), lambda i: (0, 0)),
      ],
      out_specs=pl.BlockSpec((block, n), lambda i: (i, 0)),
      out_shape=jax.ShapeDtypeStruct((m, k), jnp.float32),
  )(acc, w, b.reshape(1, n))


_FBLK = 2000
_FGRID = N // _FBLK


def _final_body(fa_ref, acc_ref, gid_ref, wot_ref, wob_ref, bo_ref,
                wf1_ref, bf1_ref, wf2_ref, bf2_ref, o_ref, mol_ref, cnt_ref):
  i = pl.program_id(0)

  @pl.when(i == 0)
  def _():
    mol_ref[...] = jnp.zeros_like(mol_ref)
    cnt_ref[...] = jnp.zeros_like(cnt_ref)

  a2 = acc_ref[0] + acc_ref[1]
  atom = jnp.maximum(
      jnp.dot(fa_ref[...], wot_ref[...], preferred_element_type=jnp.float32)
      + jnp.dot(a2, wob_ref[...], preferred_element_type=jnp.float32)
      + bo_ref[...],
      0.0,
  )
  gid = gid_ref[0, 0]
  onehot = (
      lax.broadcasted_iota(jnp.int32, (G, _FBLK), 0) == gid[None, :]
  ).astype(jnp.float32)
  mol_ref[...] += jnp.dot(onehot, atom, preferred_element_type=jnp.float32)
  cnt_ref[...] += jnp.sum(onehot, axis=1, keepdims=True)

  @pl.when(i == _FGRID - 1)
  def _():
    mol = mol_ref[...] / jnp.maximum(cnt_ref[...], 1.0)
    hid = jnp.maximum(
        jnp.dot(mol, wf1_ref[...], preferred_element_type=jnp.float32)
        + bf1_ref[...],
        0.0,
    )
    o_ref[...] = (
        jnp.dot(hid, wf2_ref[...], preferred_element_type=jnp.float32)
        + bf2_ref[...]
    )


def _final(f_atoms, acc, gid_row, w_o_t, w_o_b, b_o, w_f1, b_f1, w_f2, b_f2):
  t = w_f2.shape[1]
  return pl.pallas_call(
      _final_body,
      grid=(_FGRID,),
      in_specs=[
          pl.BlockSpec((_FBLK, DA), lambda i: (i, 0)),
          pl.BlockSpec((2, _FBLK, H), lambda i: (0, i, 0)),
          pl.BlockSpec((1, 1, _FBLK), lambda i: (i, 0, 0)),
          pl.BlockSpec((DA, H), lambda i: (0, 0)),
          pl.BlockSpec((H, H), lambda i: (0, 0)),
          pl.BlockSpec((1, H), lambda i: (0, 0)),
          pl.BlockSpec((H, H), lambda i: (0, 0)),
          pl.BlockSpec((1, H), lambda i: (0, 0)),
          pl.BlockSpec((H, t), lambda i: (0, 0)),
          pl.BlockSpec((1, t), lambda i: (0, 0)),
      ],
      out_specs=pl.BlockSpec((G, t), lambda i: (0, 0)),
      out_shape=jax.ShapeDtypeStruct((G, t), jnp.float32),
      scratch_shapes=[
          pltpu.VMEM((G, H), jnp.float32),
          pltpu.VMEM((G, 1), jnp.float32),
      ],
  )(f_atoms, acc, gid_row, w_o_t, w_o_b, b_o.reshape(1, H),
    w_f1, b_f1.reshape(1, H), w_f2, b_f2.reshape(1, t))


def _sweep_impl(write_h0, table, rows_in, idxr, zrows, accout, h0out,
                acc, idx0, idx1, rows0, rows1, gath0, gath1,
                sem_r0, sem_r1, sem_g0, sem_g1):
  c = lax.axis_index("c")
  s = lax.axis_index("s")
  w = s * NC + c

  sets = ((idx0, rows0, gath0, sem_r0, sem_g0),
          (idx1, rows1, gath1, sem_r1, sem_g1))

  pltpu.sync_copy(zrows, acc.at[pl.ds(s * NODE_PER_TILE, NODE_PER_TILE)])
  plsc.subcore_barrier()

  def stage(k, st):
    idx_v, rows_v, gath_v, sem_r, sem_g = st
    r = w + NW * k

    @pl.when(r < ROWS)
    def _():
      pltpu.sync_copy(idxr.at[r], idx_v)
      pltpu.async_copy(rows_in.at[pl.ds(r * CHUNK, CHUNK)], rows_v, sem_r)
      pltpu.async_copy(table.at[idx_v.at[0]], gath_v, sem_g)

  stage(0, sets[0])

  def outer(p, carry):
    for j in range(2):
      k = p * 2 + j
      idx_v, rows_v, gath_v, sem_r, sem_g = sets[j]
      r = w + NW * k
      stage(k + 1, sets[1 - j])

      @pl.when(r < ROWS)
      def _():
        pltpu.make_async_copy(
            rows_in.at[pl.ds(r * CHUNK, CHUNK)], rows_v, sem_r).wait()
        pltpu.make_async_copy(table.at[idx_v.at[0]], gath_v, sem_g).wait()

        def crow(rr, cy):
          for q in range(H // 16):
            sl = pl.ds(q * 16, 16)
            gath_v[rr, sl] = jnp.maximum(gath_v[rr, sl] + rows_v[rr, sl], 0.0)
          return cy

        lax.fori_loop(0, CHUNK, crow, 0)
        if write_h0:
          pltpu.sync_copy(gath_v, h0out.at[pl.ds(r * CHUNK, CHUNK)])
        pltpu.sync_copy(gath_v, acc.at[idx_v.at[1]], add=True)

    return carry

  lax.fori_loop(0, (MAX_K + 2) // 2, outer, 0)

  plsc.subcore_barrier()
  pltpu.sync_copy(
      acc.at[pl.ds(s * NODE_PER_TILE, NODE_PER_TILE)],
      accout.at[c, pl.ds(s * NODE_PER_TILE, NODE_PER_TILE)],
  )


def _sweep0_body(table, rows_in, idxr, zrows, accout, h0out, *rest):
  _sweep_impl(True, table, rows_in, idxr, zrows, accout, h0out, *rest)


def _sweep_body(table, rows_in, idxr, zrows, accout, *rest):
  _sweep_impl(False, table, rows_in, idxr, zrows, accout, None, *rest)


_SC_SCRATCH = [
    pltpu.VMEM_SHARED((NP, H), jnp.float32),
    pltpu.VMEM((2, CHUNK), jnp.int32),
    pltpu.VMEM((2, CHUNK), jnp.int32),
    pltpu.VMEM((CHUNK, H), jnp.float32),
    pltpu.VMEM((CHUNK, H), jnp.float32),
    pltpu.VMEM((CHUNK, H), jnp.float32),
    pltpu.VMEM((CHUNK, H), jnp.float32),
    pltpu.SemaphoreType.DMA,
    pltpu.SemaphoreType.DMA,
    pltpu.SemaphoreType.DMA,
    pltpu.SemaphoreType.DMA,
]

_MESH = plsc.VectorSubcoreMesh(core_axis_name="c", subcore_axis_name="s")

_sweep0 = pl.kernel(
    _sweep0_body,
    out_type=(
        jax.ShapeDtypeStruct((2, NP, H), jnp.float32),
        jax.ShapeDtypeStruct((E, H), jnp.float32),
    ),
    mesh=_MESH,
    scratch_types=_SC_SCRATCH,
)

_sweep = pl.kernel(
    _sweep_body,
    out_type=jax.ShapeDtypeStruct((2, NP, H), jnp.float32),
    mesh=_MESH,
    scratch_types=_SC_SCRATCH,
)


def kernel(f_atoms, f_bonds, edge_index, graph_ids,
           W_i, b_i, W_h, b_h, W_o, b_o, W_f1, b_f1, W_f2, b_f2):
  idxr = edge_index.astype(jnp.int32).reshape(2, ROWS, CHUNK).transpose(1, 0, 2)
  gid_row = graph_ids.astype(jnp.int32).reshape(_FGRID, 1, _FBLK)
  zrows = jnp.zeros((NODE_PER_TILE, H), jnp.float32)

  A = _dense(f_atoms, W_i[:DA], jnp.zeros((H,), jnp.float32), block=2000)
  B = _dense(f_bonds, W_i[DA:], b_i, block=2000)

  acc0, h0 = _sweep0(A, B, idxr, zrows)
  Q0 = _dense_fold(acc0, W_h, b_h, block=2000)
  acc1 = _sweep(Q0, h0, idxr, zrows)
  Q1 = _dense_fold(acc1, W_h, b_h, block=2000)
  acc2 = _sweep(Q1, h0, idxr, zrows)

  return _final(f_atoms, acc2, gid_row, W_o[:DA], W_o[DA:], b_o,
                W_f1, b_f1, W_f2, b_f2)
